# baseline (speedup 1.0000x reference)
import functools

import jax
import jax.numpy as jnp
import numpy as np
from jax import lax
from jax.experimental import pallas as pl
from jax.experimental.pallas import tpu as pltpu

_VMEM_LIMIT = 64 * 1024 * 1024
_TINY = 1e-12


def _prep_body(xt_ref, w1_ref, wcos_ref, rep_ref,
               feat_ref, xw1_ref, *, num_pers):
    xt = xt_ref[...]
    f = xt.shape[0]
    xw1_ref[...] = lax.dot_general(
        xt, w1_ref[...], dimension_numbers=(((0,), (0,)), ((), ())),
        preferred_element_type=jnp.float32).astype(jnp.bfloat16)
    w = wcos_ref[...]
    nrm_sq = lax.dot_general(xt * xt, w * w,
                             dimension_numbers=(((0,), (1,)), ((), ())),
                             preferred_element_type=jnp.float32)
    inv_nrm = lax.rsqrt(jnp.maximum(nrm_sq, 1e-24)) * (1.0 / (num_pers ** 0.5))
    xrep = lax.dot_general(xt, rep_ref[...],
                           dimension_numbers=(((0,), (0,)), ((), ())),
                           preferred_element_type=jnp.float32)
    lane = lax.broadcasted_iota(jnp.int32, (num_pers, num_pers * f), 1)
    pers = lax.broadcasted_iota(jnp.int32, (num_pers, num_pers * f), 0)
    wsel = jnp.where(lane // f == pers,
                     jnp.concatenate([w] * num_pers, axis=1), 0.0)
    scale = jnp.dot(inv_nrm, wsel,
                    preferred_element_type=jnp.float32)
    feat_ref[...] = (xrep * scale).astype(jnp.bfloat16)


def _learn_kernel(xt_ref, w1_ref, wcos_ref, rep_ref, b1_ref, w2t_ref,
                  init_adj_ref, adj_ref, raw_ref, hw2_ref,
                  feat_ref, xw1_ref, *,
                  tile_m, half, graph_skip_conn, num_pers):
    i = pl.program_id(0)

    @pl.when((i == 0) | (i == half))
    def _prep():
        _prep_body(xt_ref, w1_ref, wcos_ref, rep_ref, feat_ref, xw1_ref,
                   num_pers=num_pers)

    row0 = pl.multiple_of(i * tile_m, tile_m)
    feat_rows = feat_ref[pl.ds(row0, tile_m), :]

    att = lax.dot_general(feat_rows, feat_ref[...],
                          dimension_numbers=(((1,), (1,)), ((), ())),
                          preferred_element_type=jnp.float32)

    raw = jnp.maximum(att, 0.0)
    raw_ref[...] = raw

    row_sum = jnp.sum(raw, axis=-1, keepdims=True)
    inv_row = pl.reciprocal(jnp.maximum(row_sum, _TINY), approx=True)
    adj_tile = (graph_skip_conn * init_adj_ref[...]
                + raw * ((1.0 - graph_skip_conn) * inv_row))
    adj_ref[...] = adj_tile

    h = jnp.dot(adj_tile.astype(jnp.bfloat16), xw1_ref[...],
                preferred_element_type=jnp.float32) + b1_ref[...]
    h1 = jnp.maximum(h, 0.0)
    hw2_ref[...] = lax.dot_general(
        w2t_ref[...], h1, dimension_numbers=(((1,), (1,)), ((), ())),
        preferred_element_type=jnp.float32)


def _hop2_kernel(adj_ref, hw2_ref, b2_ref, out_ref):
    z = lax.dot_general(
        adj_ref[...], hw2_ref[...],
        dimension_numbers=(((1,), (1,)), ((), ())),
        preferred_element_type=jnp.float32) + b2_ref[...]
    zmax = jnp.max(z, axis=-1, keepdims=True)
    zs = z - zmax
    lse = jnp.log(jnp.sum(jnp.exp(zs), axis=-1, keepdims=True))
    out_ref[...] = (zs - lse).T


def kernel(node_features, init_adj, w_cos, w1, b1, w2, b2):
    n, f = node_features.shape
    num_pers = w_cos.shape[0]
    h_dim = w1.shape[1]
    c = w2.shape[1]
    k = num_pers * f
    tile_m = min(512, n)
    assert n % tile_m == 0
    n_tiles = n // tile_m
    graph_skip_conn = 0.8

    xt = node_features.T
    w2t = w2.T
    rep = jnp.asarray(np.tile(np.eye(f, dtype=np.float32),
                              (1, num_pers)))

    learn_cost = pl.CostEstimate(
        flops=2 * n * n * k + 2 * n * n * h_dim + 2 * n * h_dim * c + 8 * n * n,
        transcendentals=n,
        bytes_accessed=4 * (3 * n * n + n * h_dim + n * c) + n * k + n * h_dim,
    )
    learn = pl.pallas_call(
        functools.partial(_learn_kernel, tile_m=tile_m, half=n_tiles // 2,
                          graph_skip_conn=graph_skip_conn, num_pers=num_pers),
        grid=(n_tiles,),
        out_shape=(jax.ShapeDtypeStruct((n, n), jnp.float32),
                   jax.ShapeDtypeStruct((n, n), jnp.float32),
                   jax.ShapeDtypeStruct((c, n), jnp.float32)),
        in_specs=[
            pl.BlockSpec((f, n), lambda i: (0, 0)),
            pl.BlockSpec((f, h_dim), lambda i: (0, 0)),
            pl.BlockSpec((num_pers, f), lambda i: (0, 0)),
            pl.BlockSpec((f, k), lambda i: (0, 0)),
            pl.BlockSpec((1, h_dim), lambda i: (0, 0)),
            pl.BlockSpec((c, h_dim), lambda i: (0, 0)),
            pl.BlockSpec((tile_m, n), lambda i: (i, 0)),
        ],
        out_specs=(
            pl.BlockSpec((tile_m, n), lambda i: (i, 0)),
            pl.BlockSpec((tile_m, n), lambda i: (i, 0)),
            pl.BlockSpec((c, tile_m), lambda i: (0, i)),
        ),
        scratch_shapes=[
            pltpu.VMEM((n, k), jnp.bfloat16),
            pltpu.VMEM((n, h_dim), jnp.bfloat16),
        ],
        compiler_params=pltpu.CompilerParams(
            dimension_semantics=("parallel",),
            vmem_limit_bytes=_VMEM_LIMIT),
        cost_estimate=learn_cost,
    )
    adj, raw_adj, hw2 = learn(xt, w1, w_cos, rep, b1, w2t, init_adj)

    hop2_cost = pl.CostEstimate(
        flops=2 * n * n * c + 6 * n * c,
        transcendentals=n * (c + 1),
        bytes_accessed=4 * (n * n + n * c + c + n * c),
    )
    tile_m2 = min(256, n)
    hop2 = pl.pallas_call(
        _hop2_kernel,
        grid=(n // tile_m2,),
        out_shape=jax.ShapeDtypeStruct((c, n), jnp.float32),
        in_specs=[
            pl.BlockSpec((tile_m2, n), lambda i: (i, 0)),
            pl.BlockSpec((c, n), lambda i: (0, 0)),
            pl.BlockSpec((1, c), lambda i: (0, 0)),
        ],
        out_specs=pl.BlockSpec((c, tile_m2), lambda i: (0, i)),
        compiler_params=pltpu.CompilerParams(
            dimension_semantics=("parallel",),
            vmem_limit_bytes=_VMEM_LIMIT),
        cost_estimate=hop2_cost,
    )
    output_t = hop2(adj, hw2, b2)
    return output_t.T, adj, raw_adj


# speedup vs baseline: 1.0309x; 1.0309x over previous
import functools

import jax
import jax.numpy as jnp
import numpy as np
from jax import lax
from jax.experimental import pallas as pl
from jax.experimental.pallas import tpu as pltpu

_VMEM_LIMIT = 64 * 1024 * 1024
_TINY = 1e-12


def _prep_body(xt_ref, w1_ref, wcos_ref, rep_ref,
               feat_ref, xw1_ref, *, num_pers):
    xt = xt_ref[...]
    f = xt.shape[0]
    xw1_ref[...] = lax.dot_general(
        xt, w1_ref[...], dimension_numbers=(((0,), (0,)), ((), ())),
        preferred_element_type=jnp.float32).astype(jnp.bfloat16)
    w = wcos_ref[...]
    nrm_sq = lax.dot_general(xt * xt, w * w,
                             dimension_numbers=(((0,), (1,)), ((), ())),
                             preferred_element_type=jnp.float32)
    inv_nrm = lax.rsqrt(jnp.maximum(nrm_sq, 1e-24)) * (1.0 / (num_pers ** 0.5))
    xrep = lax.dot_general(xt, rep_ref[...],
                           dimension_numbers=(((0,), (0,)), ((), ())),
                           preferred_element_type=jnp.float32)
    lane = lax.broadcasted_iota(jnp.int32, (num_pers, num_pers * f), 1)
    pers = lax.broadcasted_iota(jnp.int32, (num_pers, num_pers * f), 0)
    wsel = jnp.where(lane // f == pers,
                     jnp.concatenate([w] * num_pers, axis=1), 0.0)
    scale = jnp.dot(inv_nrm, wsel,
                    preferred_element_type=jnp.float32)
    feat_ref[...] = (xrep * scale).astype(jnp.bfloat16)


def _learn_kernel(xt_ref, w1_ref, wcos_ref, rep_ref, b1_ref, w2t_ref,
                  init_adj_ref, adj_ref, raw_ref, hw2_ref,
                  feat_ref, xw1_ref, *,
                  tile_m, half, graph_skip_conn, num_pers):
    i = pl.program_id(0)

    @pl.when((i == 0) | (i == half))
    def _prep():
        _prep_body(xt_ref, w1_ref, wcos_ref, rep_ref, feat_ref, xw1_ref,
                   num_pers=num_pers)

    row0 = pl.multiple_of(i * tile_m, tile_m)
    feat_rows = feat_ref[pl.ds(row0, tile_m), :]

    att = lax.dot_general(feat_rows, feat_ref[...],
                          dimension_numbers=(((1,), (1,)), ((), ())),
                          preferred_element_type=jnp.float32)

    raw = jnp.maximum(att, 0.0)
    raw_ref[...] = raw

    row_sum = jnp.sum(raw, axis=-1, keepdims=True)
    inv_row = pl.reciprocal(jnp.maximum(row_sum, _TINY), approx=True)
    adj_tile = (graph_skip_conn * init_adj_ref[...]
                + raw * ((1.0 - graph_skip_conn) * inv_row))
    adj_ref[...] = adj_tile

    h = jnp.dot(adj_tile.astype(jnp.bfloat16), xw1_ref[...],
                preferred_element_type=jnp.float32) + b1_ref[...]
    h1 = jnp.maximum(h, 0.0)
    hw2_ref[...] = lax.dot_general(
        w2t_ref[...], h1, dimension_numbers=(((1,), (1,)), ((), ())),
        preferred_element_type=jnp.float32)


def _hop2_kernel(adj_ref, hw2_ref, b2_ref, out_ref):
    z = lax.dot_general(
        adj_ref[...], hw2_ref[...],
        dimension_numbers=(((1,), (1,)), ((), ())),
        preferred_element_type=jnp.float32) + b2_ref[...]
    zmax = jnp.max(z, axis=-1, keepdims=True)
    zs = z - zmax
    lse = jnp.log(jnp.sum(jnp.exp(zs), axis=-1, keepdims=True))
    out_ref[...] = (zs - lse).T


def kernel(node_features, init_adj, w_cos, w1, b1, w2, b2):
    n, f = node_features.shape
    num_pers = w_cos.shape[0]
    h_dim = w1.shape[1]
    c = w2.shape[1]
    k = num_pers * f
    tile_m = min(512, n)
    assert n % tile_m == 0
    n_tiles = n // tile_m
    graph_skip_conn = 0.8

    xt = node_features.T
    w2t = w2.T
    rep = jnp.asarray(np.tile(np.eye(f, dtype=np.float32),
                              (1, num_pers)))

    learn_cost = pl.CostEstimate(
        flops=2 * n * n * k + 2 * n * n * h_dim + 2 * n * h_dim * c + 8 * n * n,
        transcendentals=n,
        bytes_accessed=4 * (3 * n * n + n * h_dim + n * c) + n * k + n * h_dim,
    )
    learn = pl.pallas_call(
        functools.partial(_learn_kernel, tile_m=tile_m, half=n_tiles // 2,
                          graph_skip_conn=graph_skip_conn, num_pers=num_pers),
        grid=(n_tiles,),
        out_shape=(jax.ShapeDtypeStruct((n, n), jnp.float32),
                   jax.ShapeDtypeStruct((n, n), jnp.float32),
                   jax.ShapeDtypeStruct((c, n), jnp.float32)),
        in_specs=[
            pl.BlockSpec((f, n), lambda i: (0, 0)),
            pl.BlockSpec((f, h_dim), lambda i: (0, 0)),
            pl.BlockSpec((num_pers, f), lambda i: (0, 0)),
            pl.BlockSpec((f, k), lambda i: (0, 0)),
            pl.BlockSpec((1, h_dim), lambda i: (0, 0)),
            pl.BlockSpec((c, h_dim), lambda i: (0, 0)),
            pl.BlockSpec((tile_m, n), lambda i: (i, 0)),
        ],
        out_specs=(
            pl.BlockSpec((tile_m, n), lambda i: (i, 0)),
            pl.BlockSpec((tile_m, n), lambda i: (i, 0)),
            pl.BlockSpec((c, tile_m), lambda i: (0, i)),
        ),
        scratch_shapes=[
            pltpu.VMEM((n, k), jnp.bfloat16),
            pltpu.VMEM((n, h_dim), jnp.bfloat16),
        ],
        compiler_params=pltpu.CompilerParams(
            dimension_semantics=("parallel",),
            vmem_limit_bytes=_VMEM_LIMIT),
        cost_estimate=learn_cost,
    )
    adj, raw_adj, hw2 = learn(xt, w1, w_cos, rep, b1, w2t, init_adj)

    hop2_cost = pl.CostEstimate(
        flops=2 * n * n * c + 6 * n * c,
        transcendentals=n * (c + 1),
        bytes_accessed=4 * (n * n + n * c + c + n * c),
    )
    tile_m2 = min(1024, n)
    hop2 = pl.pallas_call(
        _hop2_kernel,
        grid=(n // tile_m2,),
        out_shape=jax.ShapeDtypeStruct((c, n), jnp.float32),
        in_specs=[
            pl.BlockSpec((tile_m2, n), lambda i: (i, 0)),
            pl.BlockSpec((c, n), lambda i: (0, 0)),
            pl.BlockSpec((1, c), lambda i: (0, 0)),
        ],
        out_specs=pl.BlockSpec((c, tile_m2), lambda i: (0, i)),
        compiler_params=pltpu.CompilerParams(
            dimension_semantics=("parallel",),
            vmem_limit_bytes=_VMEM_LIMIT),
        cost_estimate=hop2_cost,
    )
    output_t = hop2(adj, hw2, b2)
    return output_t.T, adj, raw_adj


# speedup vs baseline: 1.0445x; 1.0132x over previous
import functools

import jax
import jax.numpy as jnp
import numpy as np
from jax import lax
from jax.experimental import pallas as pl
from jax.experimental.pallas import tpu as pltpu

_VMEM_LIMIT = 64 * 1024 * 1024
_TINY = 1e-12


def _prep_body(xt_ref, w1_ref, wcos_ref, rep_ref,
               feat_ref, xw1_ref, *, num_pers):
    xt = xt_ref[...]
    f = xt.shape[0]
    xw1_ref[...] = lax.dot_general(
        xt, w1_ref[...], dimension_numbers=(((0,), (0,)), ((), ())),
        preferred_element_type=jnp.float32).astype(jnp.bfloat16)
    w = wcos_ref[...]
    nrm_sq = lax.dot_general(xt * xt, w * w,
                             dimension_numbers=(((0,), (1,)), ((), ())),
                             preferred_element_type=jnp.float32)
    inv_nrm = lax.rsqrt(jnp.maximum(nrm_sq, 1e-24)) * (1.0 / (num_pers ** 0.5))
    xrep = lax.dot_general(xt, rep_ref[...],
                           dimension_numbers=(((0,), (0,)), ((), ())),
                           preferred_element_type=jnp.float32)
    lane = lax.broadcasted_iota(jnp.int32, (num_pers, num_pers * f), 1)
    pers = lax.broadcasted_iota(jnp.int32, (num_pers, num_pers * f), 0)
    wsel = jnp.where(lane // f == pers,
                     jnp.concatenate([w] * num_pers, axis=1), 0.0)
    scale = jnp.dot(inv_nrm, wsel,
                    preferred_element_type=jnp.float32)
    feat_ref[...] = (xrep * scale).astype(jnp.bfloat16)


def _learn_kernel(xt_ref, w1_ref, wcos_ref, rep_ref, b1_ref, w2t_ref,
                  init_adj_ref, adj_ref, raw_ref, hw2_ref,
                  feat_ref, xw1_ref, *,
                  tile_m, half, graph_skip_conn, num_pers):
    i = pl.program_id(0)

    @pl.when((i == 0) | (i == half))
    def _prep():
        _prep_body(xt_ref, w1_ref, wcos_ref, rep_ref, feat_ref, xw1_ref,
                   num_pers=num_pers)

    row0 = pl.multiple_of(i * tile_m, tile_m)
    feat_rows = feat_ref[pl.ds(row0, tile_m), :]

    att = lax.dot_general(feat_rows, feat_ref[...],
                          dimension_numbers=(((1,), (1,)), ((), ())),
                          preferred_element_type=jnp.float32)

    raw = jnp.maximum(att, 0.0)
    raw_ref[...] = raw

    row_sum = jnp.sum(raw, axis=-1, keepdims=True)
    inv_row = pl.reciprocal(jnp.maximum(row_sum, _TINY), approx=True)
    adj_tile = (graph_skip_conn * init_adj_ref[...]
                + raw * ((1.0 - graph_skip_conn) * inv_row))
    adj_ref[...] = adj_tile

    h = jnp.dot(adj_tile.astype(jnp.bfloat16), xw1_ref[...],
                preferred_element_type=jnp.float32) + b1_ref[...]
    h1 = jnp.maximum(h, 0.0)
    hw2_ref[...] = lax.dot_general(
        w2t_ref[...], h1, dimension_numbers=(((1,), (1,)), ((), ())),
        preferred_element_type=jnp.float32)


def _hop2_kernel(adj_ref, hw2_ref, b2_ref, out_ref):
    z = lax.dot_general(
        adj_ref[...], hw2_ref[...],
        dimension_numbers=(((1,), (1,)), ((), ())),
        preferred_element_type=jnp.float32) + b2_ref[...]
    zmax = jnp.max(z, axis=-1, keepdims=True)
    zs = z - zmax
    lse = jnp.log(jnp.sum(jnp.exp(zs), axis=-1, keepdims=True))
    out_ref[...] = (zs - lse).T


def kernel(node_features, init_adj, w_cos, w1, b1, w2, b2):
    n, f = node_features.shape
    num_pers = w_cos.shape[0]
    h_dim = w1.shape[1]
    c = w2.shape[1]
    k = num_pers * f
    tile_m = min(512, n)
    assert n % tile_m == 0
    n_tiles = n // tile_m
    graph_skip_conn = 0.8

    xt = node_features.T
    w2t = w2.T
    rep = jnp.asarray(np.tile(np.eye(f, dtype=np.float32),
                              (1, num_pers)))

    learn_cost = pl.CostEstimate(
        flops=2 * n * n * k + 2 * n * n * h_dim + 2 * n * h_dim * c + 8 * n * n,
        transcendentals=n,
        bytes_accessed=4 * (3 * n * n + n * h_dim + n * c) + n * k + n * h_dim,
    )
    learn = pl.pallas_call(
        functools.partial(_learn_kernel, tile_m=tile_m, half=n_tiles // 2,
                          graph_skip_conn=graph_skip_conn, num_pers=num_pers),
        grid=(n_tiles,),
        out_shape=(jax.ShapeDtypeStruct((n, n), jnp.float32),
                   jax.ShapeDtypeStruct((n, n), jnp.float32),
                   jax.ShapeDtypeStruct((c, n), jnp.float32)),
        in_specs=[
            pl.BlockSpec((f, n), lambda i: (0, 0)),
            pl.BlockSpec((f, h_dim), lambda i: (0, 0)),
            pl.BlockSpec((num_pers, f), lambda i: (0, 0)),
            pl.BlockSpec((f, k), lambda i: (0, 0)),
            pl.BlockSpec((1, h_dim), lambda i: (0, 0)),
            pl.BlockSpec((c, h_dim), lambda i: (0, 0)),
            pl.BlockSpec((tile_m, n), lambda i: (i, 0)),
        ],
        out_specs=(
            pl.BlockSpec((tile_m, n), lambda i: (i, 0)),
            pl.BlockSpec((tile_m, n), lambda i: (i, 0)),
            pl.BlockSpec((c, tile_m), lambda i: (0, i)),
        ),
        scratch_shapes=[
            pltpu.VMEM((n, k), jnp.bfloat16),
            pltpu.VMEM((n, h_dim), jnp.bfloat16),
        ],
        compiler_params=pltpu.CompilerParams(
            dimension_semantics=("arbitrary",),
            vmem_limit_bytes=_VMEM_LIMIT),
        cost_estimate=learn_cost,
    )
    adj, raw_adj, hw2 = learn(xt, w1, w_cos, rep, b1, w2t, init_adj)

    hop2_cost = pl.CostEstimate(
        flops=2 * n * n * c + 6 * n * c,
        transcendentals=n * (c + 1),
        bytes_accessed=4 * (n * n + n * c + c + n * c),
    )
    tile_m2 = min(512, n)
    hop2 = pl.pallas_call(
        _hop2_kernel,
        grid=(n // tile_m2,),
        out_shape=jax.ShapeDtypeStruct((c, n), jnp.float32),
        in_specs=[
            pl.BlockSpec((tile_m2, n), lambda i: (i, 0)),
            pl.BlockSpec((c, n), lambda i: (0, 0)),
            pl.BlockSpec((1, c), lambda i: (0, 0)),
        ],
        out_specs=pl.BlockSpec((c, tile_m2), lambda i: (0, i)),
        compiler_params=pltpu.CompilerParams(
            dimension_semantics=("parallel",),
            vmem_limit_bytes=_VMEM_LIMIT),
        cost_estimate=hop2_cost,
    )
    output_t = hop2(adj, hw2, b2)
    return output_t.T, adj, raw_adj
